# baseline (speedup 1.0000x reference)
import functools
import math

import jax
import jax.numpy as jnp
from jax import lax
from jax.experimental import pallas as pl
from jax.experimental.pallas import tpu as pltpu
from jax.experimental.pallas import tpu_sc as plsc

B = 1024
V = 100000
M = 0.5
COS_M = math.cos(M)
SIN_M = math.sin(M)


_NC = 2
_NS = 16
_NW = _NC * _NS
_STRIPE = 1536
_SCV = _STRIPE * _NW
_CR = 48
_NCHK = _STRIPE // _CR
_NCOL = B // 16


@functools.cache
def _build_sc_part():
    mesh = plsc.VectorSubcoreMesh(core_axis_name="c", subcore_axis_name="s")

    @functools.partial(
        pl.kernel,
        mesh=mesh,
        out_type=(
            jax.ShapeDtypeStruct((_NW, B), jnp.float32),
            jax.ShapeDtypeStruct((_NW, B), jnp.float32),
        ),
        scratch_types=[
            pltpu.VMEM((B,), jnp.int32),
            pltpu.VMEM((B,), jnp.float32),
            pltpu.VMEM((B,), jnp.float32),
            pltpu.VMEM((_CR, B), jnp.float32),
            pltpu.VMEM((_CR, B), jnp.float32),
            pltpu.SemaphoreType.DMA,
            pltpu.SemaphoreType.DMA,
        ],
    )
    def sc_kernel(xt_hbm, tgt_hbm, s_out, c_out,
                  tgt_v, acc_v, c_v, buf_a, buf_b, sem_a, sem_b):
        wid = lax.axis_index("s") * _NC + lax.axis_index("c")
        stripe0 = wid * _STRIPE
        pltpu.sync_copy(tgt_hbm, tgt_v)
        zero16 = jnp.zeros((16,), jnp.float32)

        def zbody(z, carry):
            acc_v[pl.ds(z * 16, 16)] = zero16
            c_v[pl.ds(z * 16, 16)] = zero16
            return carry

        lax.fori_loop(0, _NCOL, zbody, 0)

        def start(ch, buf, sem):
            pltpu.make_async_copy(
                xt_hbm.at[pl.ds(stripe0 + ch * _CR, _CR), :], buf, sem).start()

        def wait(ch, buf, sem):
            pltpu.make_async_copy(
                xt_hbm.at[pl.ds(stripe0 + ch * _CR, _CR), :], buf, sem).wait()

        def process(buf, gbase):
            def jbody(j, carry):
                js = pl.ds(j * 16, 16)
                trel = tgt_v[js] - gbase
                a = acc_v[js]
                c = c_v[js]
                for i in range(_CR):
                    v = buf[i, js]
                    a = a + jnp.exp(v)
                    c = jnp.where(trel == i, v, c)
                acc_v[js] = a
                c_v[js] = c
                return carry
            lax.fori_loop(0, _NCOL, jbody, 0)

        start(0, buf_a, sem_a)

        def pair_body(p, carry):
            start(2 * p + 1, buf_b, sem_b)
            wait(2 * p, buf_a, sem_a)
            process(buf_a, stripe0 + 2 * p * _CR)

            @pl.when(p + 1 < _NCHK // 2)
            def _next():
                start(2 * p + 2, buf_a, sem_a)

            wait(2 * p + 1, buf_b, sem_b)
            process(buf_b, stripe0 + (2 * p + 1) * _CR)
            return carry

        lax.fori_loop(0, _NCHK // 2, pair_body, 0)

        pltpu.sync_copy(acc_v, s_out.at[wid])
        pltpu.sync_copy(c_v, c_out.at[wid])

    return sc_kernel


_VB = 2048
_VB0 = _SCV // _VB
_NBT = -(-(V - _SCV) // _VB)


def _tc_body(xt_ref, tgt_ref, sp_ref, cp_ref, out_ref, acc_ref, cacc_ref):
    i = pl.program_id(0)

    @pl.when(i == 0)
    def _init():
        acc_ref[...] = jnp.zeros_like(acc_ref)
        cacc_ref[...] = jnp.zeros_like(cacc_ref)

    rowbase = (_VB0 + i) * _VB
    tvec = tgt_ref[...]
    acc = acc_ref[...]
    cacc = cacc_ref[...]
    for k in range(_VB // 8):
        xs = xt_ref[k * 8:(k + 1) * 8, :]
        rid = (lax.broadcasted_iota(jnp.int32, (8, B), 0)
               + (rowbase + k * 8))
        acc += jnp.where(rid < V, jnp.exp(xs), 0.0)
        cacc += jnp.where(rid == tvec, xs, 0.0)
    acc_ref[...] = acc
    cacc_ref[...] = cacc

    @pl.when(i == _NBT - 1)
    def _finish():
        s = jnp.sum(acc_ref[...], axis=0, keepdims=True)
        c = jnp.sum(cacc_ref[...], axis=0, keepdims=True)
        s += jnp.sum(sp_ref[...], axis=0, keepdims=True)
        c += jnp.sum(cp_ref[...], axis=0, keepdims=True)
        sin_t = jnp.sqrt(jnp.maximum(1.0 - c * c, 0.0))
        new_cos = c * COS_M - sin_t * SIN_M
        stot = s - jnp.exp(c) + jnp.exp(new_cos)
        nll = jnp.log(stot) - new_cos
        out_ref[0, 0] = jnp.sum(nll) / B


def _tc_loss(xt, tgt, s_part, c_part):
    return pl.pallas_call(
        _tc_body,
        grid=(_NBT,),
        in_specs=[
            pl.BlockSpec((_VB, B), lambda i: (_VB0 + i, 0)),
            pl.BlockSpec((1, B), lambda i: (0, 0)),
            pl.BlockSpec((_NW, B), lambda i: (0, 0)),
            pl.BlockSpec((_NW, B), lambda i: (0, 0)),
        ],
        out_specs=pl.BlockSpec(memory_space=pltpu.SMEM),
        out_shape=jax.ShapeDtypeStruct((1, 1), jnp.float32),
        scratch_shapes=[
            pltpu.VMEM((8, B), jnp.float32),
            pltpu.VMEM((8, B), jnp.float32),
        ],
    )(xt, tgt, s_part, c_part)


def kernel(input, target):
    xt = input.T
    tgt = target.astype(jnp.int32)
    s_part, c_part = _build_sc_part()(xt, tgt)
    out = _tc_loss(xt, tgt.reshape(1, B), s_part, c_part)
    return out[0, 0]


# speedup vs baseline: 1.0819x; 1.0819x over previous
import functools
import math

import jax
import jax.numpy as jnp
from jax import lax
from jax.experimental import pallas as pl
from jax.experimental.pallas import tpu as pltpu
from jax.experimental.pallas import tpu_sc as plsc

B = 1024
V = 100000
M = 0.5
COS_M = math.cos(M)
SIN_M = math.sin(M)


_NC = 2
_NS = 16
_NW = _NC * _NS
_STRIPE = 1024
_SCV = _STRIPE * _NW
_CR = 32
_NCHK = _STRIPE // _CR
_NCOL = B // 16


@functools.cache
def _build_sc_part():
    mesh = plsc.VectorSubcoreMesh(core_axis_name="c", subcore_axis_name="s")

    @functools.partial(
        pl.kernel,
        mesh=mesh,
        out_type=(
            jax.ShapeDtypeStruct((_NW, B), jnp.float32),
            jax.ShapeDtypeStruct((_NW, B), jnp.float32),
        ),
        scratch_types=[
            pltpu.VMEM((B,), jnp.int32),
            pltpu.VMEM((B,), jnp.float32),
            pltpu.VMEM((B,), jnp.float32),
            pltpu.VMEM((_CR, B), jnp.float32),
            pltpu.VMEM((_CR, B), jnp.float32),
            pltpu.SemaphoreType.DMA,
            pltpu.SemaphoreType.DMA,
        ],
    )
    def sc_kernel(xt_hbm, tgt_hbm, s_out, c_out,
                  tgt_v, acc_v, c_v, buf_a, buf_b, sem_a, sem_b):
        wid = lax.axis_index("s") * _NC + lax.axis_index("c")
        stripe0 = wid * _STRIPE
        pltpu.sync_copy(tgt_hbm, tgt_v)
        zero16 = jnp.zeros((16,), jnp.float32)

        def zbody(z, carry):
            acc_v[pl.ds(z * 16, 16)] = zero16
            c_v[pl.ds(z * 16, 16)] = zero16
            return carry

        lax.fori_loop(0, _NCOL, zbody, 0)

        def start(ch, buf, sem):
            pltpu.make_async_copy(
                xt_hbm.at[pl.ds(stripe0 + ch * _CR, _CR), :], buf, sem).start()

        def wait(ch, buf, sem):
            pltpu.make_async_copy(
                xt_hbm.at[pl.ds(stripe0 + ch * _CR, _CR), :], buf, sem).wait()

        def process(buf, gbase):
            def jbody(j, carry):
                js = pl.ds(j * 16, 16)
                trel = tgt_v[js] - gbase
                a = acc_v[js]
                c = c_v[js]
                for i in range(_CR):
                    v = buf[i, js]
                    a = a + jnp.exp(v)
                    c = jnp.where(trel == i, v, c)
                acc_v[js] = a
                c_v[js] = c
                return carry
            lax.fori_loop(0, _NCOL, jbody, 0)

        start(0, buf_a, sem_a)

        def pair_body(p, carry):
            start(2 * p + 1, buf_b, sem_b)
            wait(2 * p, buf_a, sem_a)
            process(buf_a, stripe0 + 2 * p * _CR)

            @pl.when(p + 1 < _NCHK // 2)
            def _next():
                start(2 * p + 2, buf_a, sem_a)

            wait(2 * p + 1, buf_b, sem_b)
            process(buf_b, stripe0 + (2 * p + 1) * _CR)
            return carry

        lax.fori_loop(0, _NCHK // 2, pair_body, 0)

        pltpu.sync_copy(acc_v, s_out.at[wid])
        pltpu.sync_copy(c_v, c_out.at[wid])

    return sc_kernel


_VB = 2048
_VB0 = _SCV // _VB
_NBT = -(-(V - _SCV) // _VB)


def _tc_body(xt_ref, tgt_ref, sp_ref, cp_ref, out_ref, acc_ref, cacc_ref):
    i = pl.program_id(0)

    @pl.when(i == 0)
    def _init():
        acc_ref[...] = jnp.zeros_like(acc_ref)
        cacc_ref[...] = jnp.zeros_like(cacc_ref)

    rowbase = (_VB0 + i) * _VB
    tvec = tgt_ref[...]
    acc = acc_ref[...]
    cacc = cacc_ref[...]
    for k in range(_VB // 8):
        xs = xt_ref[k * 8:(k + 1) * 8, :]
        rid = (lax.broadcasted_iota(jnp.int32, (8, B), 0)
               + (rowbase + k * 8))
        acc += jnp.where(rid < V, jnp.exp(xs), 0.0)
        cacc += jnp.where(rid == tvec, xs, 0.0)
    acc_ref[...] = acc
    cacc_ref[...] = cacc

    @pl.when(i == _NBT - 1)
    def _finish():
        s = jnp.sum(acc_ref[...], axis=0, keepdims=True)
        c = jnp.sum(cacc_ref[...], axis=0, keepdims=True)
        s += jnp.sum(sp_ref[...], axis=0, keepdims=True)
        c += jnp.sum(cp_ref[...], axis=0, keepdims=True)
        sin_t = jnp.sqrt(jnp.maximum(1.0 - c * c, 0.0))
        new_cos = c * COS_M - sin_t * SIN_M
        stot = s - jnp.exp(c) + jnp.exp(new_cos)
        nll = jnp.log(stot) - new_cos
        out_ref[0, 0] = jnp.sum(nll) / B


def _tc_loss(xt, tgt, s_part, c_part):
    return pl.pallas_call(
        _tc_body,
        grid=(_NBT,),
        in_specs=[
            pl.BlockSpec((_VB, B), lambda i: (_VB0 + i, 0)),
            pl.BlockSpec((1, B), lambda i: (0, 0)),
            pl.BlockSpec((_NW, B), lambda i: (0, 0)),
            pl.BlockSpec((_NW, B), lambda i: (0, 0)),
        ],
        out_specs=pl.BlockSpec(memory_space=pltpu.SMEM),
        out_shape=jax.ShapeDtypeStruct((1, 1), jnp.float32),
        scratch_shapes=[
            pltpu.VMEM((8, B), jnp.float32),
            pltpu.VMEM((8, B), jnp.float32),
        ],
    )(xt, tgt, s_part, c_part)


def kernel(input, target):
    xt = input.T
    tgt = target.astype(jnp.int32)
    s_part, c_part = _build_sc_part()(xt, tgt)
    out = _tc_loss(xt, tgt.reshape(1, B), s_part, c_part)
    return out[0, 0]
